# Initial kernel scaffold; baseline (speedup 1.0000x reference)
import jax
import jax.numpy as jnp
from jax.experimental import pallas as pl


def kernel(x, pos, edge_index, W1, b1, W2, b2, W3, b3, W4, b4, W5, b5, W6, b6):
    raise NotImplementedError("write your pallas kernel here")


# speedup vs baseline: 4.6225x; 4.6225x over previous
import functools

import jax
import jax.numpy as jnp
from jax import lax
from jax.experimental import pallas as pl
from jax.experimental.pallas import tpu as pltpu
from jax.experimental.pallas import tpu_sc as plsc

N = 10000
E = 320000
D = 128
MSG = 16
PPAD = 16
MROW = 32

NC = 2
NS = 16
NW = NC * NS
PER_W = E // NW
G_CH = 400
S_CH = 1000

def _silu(v):
    return v * jax.nn.sigmoid(v)


@functools.lru_cache(maxsize=None)
def _get_gather_kernel():
    mesh = plsc.VectorSubcoreMesh(core_axis_name="c", subcore_axis_name="s")

    @functools.partial(
        pl.kernel,
        out_type=(
            jax.ShapeDtypeStruct((E, D), jnp.float32),
            jax.ShapeDtypeStruct((E, D), jnp.float32),
            jax.ShapeDtypeStruct((E, PPAD), jnp.float32),
            jax.ShapeDtypeStruct((E, PPAD), jnp.float32),
        ),
        mesh=mesh,
        scratch_types=[
            pltpu.VMEM((G_CH,), jnp.int32),
            pltpu.VMEM((G_CH,), jnp.int32),
            pltpu.VMEM((G_CH, D), jnp.float32),
            pltpu.VMEM((G_CH, D), jnp.float32),
            pltpu.VMEM((G_CH, PPAD), jnp.float32),
            pltpu.VMEM((G_CH, PPAD), jnp.float32),
            pltpu.SemaphoreType.DMA,
        ],
        compiler_params=pltpu.CompilerParams(use_tc_tiling_on_sc=False),
    )
    def _gather_kernel(tx, tp, dst, src, gxd, gxs, gpd, gps,
                       idx_d, idx_s, xb_d, xb_s, pb_d, pb_s, sem):
        w = lax.axis_index("s") * NC + lax.axis_index("c")
        base = w * PER_W

        @pl.loop(0, PER_W, step=G_CH)
        def _(off):
            e0 = base + off
            pltpu.sync_copy(dst.at[pl.ds(e0, G_CH)], idx_d)
            pltpu.sync_copy(src.at[pl.ds(e0, G_CH)], idx_s)
            c1 = pltpu.async_copy(tx.at[idx_d], xb_d, sem)
            c2 = pltpu.async_copy(tx.at[idx_s], xb_s, sem)
            c3 = pltpu.async_copy(tp.at[idx_d], pb_d, sem)
            c4 = pltpu.async_copy(tp.at[idx_s], pb_s, sem)
            c1.wait()
            c2.wait()
            c3.wait()
            c4.wait()
            pltpu.sync_copy(xb_d, gxd.at[pl.ds(e0, G_CH)])
            pltpu.sync_copy(xb_s, gxs.at[pl.ds(e0, G_CH)])
            pltpu.sync_copy(pb_d, gpd.at[pl.ds(e0, G_CH)])
            pltpu.sync_copy(pb_s, gps.at[pl.ds(e0, G_CH)])

    return _gather_kernel


EB = 2000


def _edge_body(gxd, gxs, gpd, gps, w1a, w1b, w1c, b1, w2, b2, w5, b5, w6, b6,
               out):
    rel = gpd[...] - gps[...]
    d2 = jnp.sum(rel * rel, axis=1, keepdims=True)
    xd = gxd[...].astype(jnp.bfloat16)
    xs = gxs[...].astype(jnp.bfloat16)
    acc = lax.dot_general(xd, w1a[...], (((1,), (0,)), ((), ())),
                          preferred_element_type=jnp.float32)
    acc = acc + lax.dot_general(xs, w1b[...], (((1,), (0,)), ((), ())),
                                preferred_element_type=jnp.float32)
    hpre = acc + d2 * w1c[...] + b1[...]
    h = _silu(hpre).astype(jnp.bfloat16)
    mpre = lax.dot_general(h, w2[...], (((1,), (0,)), ((), ())),
                           preferred_element_type=jnp.float32) + b2[...]
    msg = _silu(mpre)
    t = _silu(jnp.dot(msg, w5[...], preferred_element_type=jnp.float32)
              + b5[...])
    cw = jnp.dot(t, w6[...], preferred_element_type=jnp.float32) + b6[...]
    out[:, 0:MSG] = msg
    out[:, MSG:MROW] = rel * cw


def _edge_mlp(gxd, gxs, gpd, gps, w1a, w1b, w1c, b1, w2, b2, w5, b5, w6, b6):
    ein = 2 * D + 1
    const = lambda shape: pl.BlockSpec(shape, lambda i: (0, 0))
    return pl.pallas_call(
        _edge_body,
        grid=(E // EB,),
        in_specs=[
            pl.BlockSpec((EB, D), lambda i: (i, 0)),
            pl.BlockSpec((EB, D), lambda i: (i, 0)),
            pl.BlockSpec((EB, PPAD), lambda i: (i, 0)),
            pl.BlockSpec((EB, PPAD), lambda i: (i, 0)),
            const((D, ein)),
            const((D, ein)),
            const((1, ein)),
            const((1, ein)),
            const((ein, MSG)),
            const((1, MSG)),
            const((MSG, 2 * MSG)),
            const((1, 2 * MSG)),
            const((2 * MSG, 1)),
            const((1, 1)),
        ],
        out_specs=pl.BlockSpec((EB, MROW), lambda i: (i, 0)),
        out_shape=jax.ShapeDtypeStruct((E, MROW), jnp.float32),
    )(gxd, gxs, gpd, gps, w1a, w1b, w1c, b1, w2, b2, w5, b5, w6, b6)


@functools.lru_cache(maxsize=None)
def _get_scatter_kernel():
    mesh = plsc.VectorSubcoreMesh(core_axis_name="c", subcore_axis_name="s")

    @functools.partial(
        pl.kernel,
        out_type=jax.ShapeDtypeStruct((NC, N, MROW), jnp.float32),
        mesh=mesh,
        scratch_types=[
            pltpu.VMEM((S_CH,), jnp.int32),
            pltpu.VMEM((S_CH, MROW), jnp.float32),
            pltpu.VMEM_SHARED((N, MROW), jnp.float32),
            pltpu.SemaphoreType.DMA,
        ],
        compiler_params=pltpu.CompilerParams(use_tc_tiling_on_sc=False),
    )
    def _scatter_kernel(m, dst, zeros, out, idx_v, rows_v, accum, sem):
        c = lax.axis_index("c")
        s = lax.axis_index("s")
        w = s * NC + c
        base = w * PER_W

        @pl.when(s == 0)
        def _():
            pltpu.sync_copy(zeros, accum)

        plsc.subcore_barrier()

        @pl.loop(0, PER_W, step=S_CH)
        def _(off):
            e0 = base + off
            pltpu.sync_copy(dst.at[pl.ds(e0, S_CH)], idx_v)
            pltpu.sync_copy(m.at[pl.ds(e0, S_CH)], rows_v)
            pltpu.sync_copy(rows_v, accum.at[idx_v], add=True)

        plsc.subcore_barrier()

        @pl.when(s == 0)
        def _():
            pltpu.sync_copy(accum, out.at[c])

    return _scatter_kernel


NB = 2000


def _node_body(x, pos, p0, p1, w3a, w3b, b3, w4, b4, out_x, out_pos):
    agg = p0[...] + p1[...]
    am = agg[:, 0:MSG]
    ap = agg[:, MSG:MSG + 3]
    h1 = (jnp.dot(x[...], w3a[...], preferred_element_type=jnp.float32)
          + jnp.dot(am, w3b[...], preferred_element_type=jnp.float32)
          + b3[...])
    out_x[...] = (jnp.dot(_silu(h1), w4[...],
                          preferred_element_type=jnp.float32) + b4[...])
    out_pos[...] = pos[...] + ap


def _node_mlp(x, pos, p0, p1, w3a, w3b, b3, w4, b4):
    const = lambda shape: pl.BlockSpec(shape, lambda i: (0, 0))
    return pl.pallas_call(
        _node_body,
        grid=(N // NB,),
        in_specs=[
            pl.BlockSpec((NB, D), lambda i: (i, 0)),
            pl.BlockSpec((NB, 3), lambda i: (i, 0)),
            pl.BlockSpec((NB, MROW), lambda i: (i, 0)),
            pl.BlockSpec((NB, MROW), lambda i: (i, 0)),
            const((D, D)),
            const((MSG, D)),
            const((1, D)),
            const((D, D)),
            const((1, D)),
        ],
        out_specs=[
            pl.BlockSpec((NB, D), lambda i: (i, 0)),
            pl.BlockSpec((NB, 3), lambda i: (i, 0)),
        ],
        out_shape=[
            jax.ShapeDtypeStruct((N, D), jnp.float32),
            jax.ShapeDtypeStruct((N, 3), jnp.float32),
        ],
    )(x, pos, p0, p1, w3a, w3b, b3, w4, b4)


def kernel(x, pos, edge_index, W1, b1, W2, b2, W3, b3, W4, b4, W5, b5, W6, b6):
    src = edge_index[0].astype(jnp.int32)
    dst = edge_index[1].astype(jnp.int32)

    tp = jnp.pad(pos, ((0, 0), (0, PPAD - 3)))

    gxd, gxs, gpd, gps = _get_gather_kernel()(x, tp, dst, src)

    w1a = W1[:D].astype(jnp.bfloat16)
    w1b = W1[D:2 * D].astype(jnp.bfloat16)
    w1c = W1[2 * D:]
    m = _edge_mlp(gxd, gxs, gpd, gps, w1a, w1b, w1c, b1[None, :],
                  W2.astype(jnp.bfloat16), b2[None, :], W5, b5[None, :],
                  W6, b6[None, :])

    partials = _get_scatter_kernel()(m, dst, jnp.zeros((N, MROW), jnp.float32))

    out_x, out_pos = _node_mlp(x, pos, partials[0], partials[1],
                               W3[:D], W3[D:], b3[None, :], W4, b4[None, :])
    return (out_x, out_pos)
